# baseline (speedup 1.0000x reference)
import functools

import jax
import jax.numpy as jnp
import numpy as np
from jax.experimental import pallas as pl
from jax.experimental.pallas import tpu as pltpu
from jax.sharding import Mesh, PartitionSpec as P

try:
    from jax.experimental.shard_map import shard_map as _shard_map_impl
except ImportError:
    _shard_map_impl = jax.shard_map


def _shard_map(f, mesh, in_specs, out_specs):
    try:
        return _shard_map_impl(f, mesh=mesh, in_specs=in_specs,
                               out_specs=out_specs, check_rep=False)
    except TypeError:
        return _shard_map_impl(f, mesh=mesh, in_specs=in_specs,
                               out_specs=out_specs, check_vma=False)

N = 4096
NFEAT = 256
NHID = 256
OUT_DIM = 128
BN_EPS = 1e-5

I_BLK = 512


def _conv_body(a_ref, xb_ref, xres_ref, w1_ref, b1_ref, w2_ref, b2_ref,
               *out_refs, relu_out, dual_out):
    mask = (a_ref[...] > 0.0).astype(jnp.bfloat16)
    aggr = jax.lax.dot_general(
        mask, xb_ref[...], (((0,), (0,)), ((), ())),
        preferred_element_type=jnp.float32)
    aggr = aggr + xres_ref[...]
    h = jnp.dot(aggr.astype(jnp.bfloat16), w1_ref[...],
                preferred_element_type=jnp.float32) + b1_ref[...]
    h = jnp.maximum(h, 0.0)
    o = jnp.dot(h.astype(jnp.bfloat16), w2_ref[...],
                preferred_element_type=jnp.float32) + b2_ref[...]
    if relu_out:
        o = jnp.maximum(o, 0.0)
    out_refs[0][...] = o
    if dual_out:
        out_refs[1][...] = o.astype(jnp.bfloat16)


def _gin_conv(A, xb, xres, w1, b1, w2, b2, out_dim, relu_out, dual_out):
    n_cols = A.shape[1]
    n_i = n_cols // I_BLK
    full = lambda shape: pl.BlockSpec(shape, lambda i: (0, 0))
    in_specs = [
        pl.BlockSpec((N, I_BLK), lambda i: (0, i)),
        full((N, NFEAT)),
        pl.BlockSpec((I_BLK, NFEAT), lambda i: (i, 0)),
        full(w1.shape),
        full(b1.shape),
        full(w2.shape),
        full(b2.shape),
    ]
    out_shape = [jax.ShapeDtypeStruct((n_cols, out_dim), jnp.float32)]
    out_specs = [pl.BlockSpec((I_BLK, out_dim), lambda i: (i, 0))]
    if dual_out:
        out_shape.append(jax.ShapeDtypeStruct((n_cols, out_dim), jnp.bfloat16))
        out_specs.append(pl.BlockSpec((I_BLK, out_dim), lambda i: (i, 0)))
    return pl.pallas_call(
        functools.partial(_conv_body, relu_out=relu_out, dual_out=dual_out),
        grid=(n_i,),
        in_specs=in_specs,
        out_specs=out_specs,
        out_shape=out_shape,
    )(A, xb, xres, w1, b1, w2, b2)


def kernel(x, A, W1a, b1a, g1a, be1a, W2a, b2a, W1b, b1b, g1b, be1b, W2b, b2b):
    inv = np.float32(1.0 / np.sqrt(1.0 + BN_EPS))
    gs_a = g1a * inv
    w1a = (W1a * gs_a[None, :]).astype(jnp.bfloat16)
    c1a = (b1a * gs_a + be1a)[None, :]
    gs_b = g1b * inv
    w1b = (W1b * gs_b[None, :]).astype(jnp.bfloat16)
    c1b = (b1b * gs_b + be1b)[None, :]
    w2a = W2a.astype(jnp.bfloat16)
    w2b = W2b.astype(jnp.bfloat16)

    xb = x.astype(jnp.bfloat16)
    c2a = b2a[None, :]
    c2b = b2b[None, :]

    devs = jax.devices()
    ndev = len(devs)
    if ndev > 1 and N % (I_BLK * ndev) == 0:
        shard = N // ndev
        mesh = Mesh(np.asarray(devs), ("i",))

        def _sharded(A_loc, x_full, xb_full, w1a_, c1a_, w2a_, c2a_,
                     w1b_, c1b_, w2b_, c2b_):
            d = jax.lax.axis_index("i")
            xres = jax.lax.dynamic_slice_in_dim(x_full, d * shard, shard, 0)
            H_loc, Hb_loc = _gin_conv(A_loc, xb_full, xres, w1a_, c1a_,
                                      w2a_, c2a_, out_dim=NHID,
                                      relu_out=True, dual_out=True)
            Hb = jax.lax.all_gather(Hb_loc, "i", axis=0, tiled=True)
            out_loc, = _gin_conv(A_loc, Hb, H_loc, w1b_, c1b_, w2b_, c2b_,
                                 out_dim=OUT_DIM, relu_out=False,
                                 dual_out=False)
            return out_loc

        rep = P(None, None)
        return _shard_map(
            _sharded, mesh=mesh,
            in_specs=(P(None, "i"),) + (rep,) * 10,
            out_specs=P("i", None))(
                A, x, xb, w1a, c1a, w2a, c2a, w1b, c1b, w2b, c2b)

    H, Hb = _gin_conv(A, xb, x, w1a, c1a, w2a, c2a,
                      out_dim=NHID, relu_out=True, dual_out=True)
    out, = _gin_conv(A, Hb, H, w1b, c1b, w2b, c2b,
                     out_dim=OUT_DIM, relu_out=False, dual_out=False)
    return out


# speedup vs baseline: 4.0258x; 4.0258x over previous
import functools

import jax
import jax.numpy as jnp
import numpy as np
from jax.experimental import pallas as pl
from jax.experimental.pallas import tpu as pltpu

N = 4096
NFEAT = 256
NHID = 256
OUT_DIM = 128
BN_EPS = 1e-5

K_BLK = 512


def _conv_body(a_ref, lhsb_ref, res_ref, w1_ref, c1_ref, w2_ref, c2_ref,
               *refs, relu_out, dual_out, n_k):
    acc_ref = refs[-1]
    out_refs = refs[:-1]
    k = pl.program_id(0)
    mask = (a_ref[...] > 0.0).astype(jnp.bfloat16)
    part = jnp.dot(lhsb_ref[...], mask, preferred_element_type=jnp.float32)

    @pl.when(k == 0)
    def _():
        acc_ref[...] = res_ref[...] + part

    @pl.when(k != 0)
    def _():
        acc_ref[...] += part

    @pl.when(k == n_k - 1)
    def _():
        aggr = acc_ref[...]
        h = jnp.dot(w1_ref[...], aggr.astype(jnp.bfloat16),
                    preferred_element_type=jnp.float32) + c1_ref[...]
        h = jnp.maximum(h, 0.0)
        o = jnp.dot(w2_ref[...], h.astype(jnp.bfloat16),
                    preferred_element_type=jnp.float32) + c2_ref[...]
        if relu_out:
            o = jnp.maximum(o, 0.0)
        out_refs[0][...] = o
        if dual_out:
            out_refs[1][...] = o.astype(jnp.bfloat16)


def _gin_conv_t(A, lhsb, res, w1, c1, w2, c2, out_dim, relu_out, dual_out):
    n_k = N // K_BLK
    full = lambda shape: pl.BlockSpec(shape, lambda k: (0, 0))
    in_specs = [
        pl.BlockSpec((K_BLK, N), lambda k: (k, 0)),
        pl.BlockSpec((NFEAT, K_BLK), lambda k: (0, k)),
        full((NFEAT, N)),
        full(w1.shape),
        full(c1.shape),
        full(w2.shape),
        full(c2.shape),
    ]
    out_shape = [jax.ShapeDtypeStruct((out_dim, N), jnp.float32)]
    out_specs = [full((out_dim, N))]
    if dual_out:
        out_shape.append(jax.ShapeDtypeStruct((out_dim, N), jnp.bfloat16))
        out_specs.append(full((out_dim, N)))
    return pl.pallas_call(
        functools.partial(_conv_body, relu_out=relu_out, dual_out=dual_out,
                          n_k=n_k),
        grid=(n_k,),
        in_specs=in_specs,
        out_specs=out_specs,
        out_shape=out_shape,
        scratch_shapes=[pltpu.VMEM((NFEAT, N), jnp.float32)],
    )(A, lhsb, res, w1, c1, w2, c2)


def kernel(x, A, W1a, b1a, g1a, be1a, W2a, b2a, W1b, b1b, g1b, be1b, W2b, b2b):
    inv = np.float32(1.0 / np.sqrt(1.0 + BN_EPS))
    gs_a = g1a * inv
    w1a = (W1a * gs_a[None, :]).T.astype(jnp.bfloat16)
    c1a = (b1a * gs_a + be1a)[:, None]
    gs_b = g1b * inv
    w1b = (W1b * gs_b[None, :]).T.astype(jnp.bfloat16)
    c1b = (b1b * gs_b + be1b)[:, None]
    w2a = W2a.T.astype(jnp.bfloat16)
    w2b = W2b.T.astype(jnp.bfloat16)
    c2a = b2a[:, None]
    c2b = b2b[:, None]

    xT = x.T
    xTb = xT.astype(jnp.bfloat16)
    HT, HTb = _gin_conv_t(A, xTb, xT, w1a, c1a, w2a, c2a,
                          out_dim=NHID, relu_out=True, dual_out=True)
    outT, = _gin_conv_t(A, HTb, HT, w1b, c1b, w2b, c2b,
                        out_dim=OUT_DIM, relu_out=False, dual_out=False)
    return outT.T
